# speedup vs baseline: 1.2069x; 1.2069x over previous
import functools

import jax
import jax.numpy as jnp
from jax import lax
from jax.experimental import pallas as pl
from jax.experimental.pallas import tpu as pltpu
from jax.experimental.pallas import tpu_sc as plsc

BATCH = 16384
VALUE_RANGE = 8.0
A_RANGE = 3.0
DCONST = 1.702

_info = plsc.get_sparse_core_info()
_NC, _NS, _L = _info.num_cores, _info.num_subcores, _info.num_lanes
_NW = _NC * _NS
_CHUNK = BATCH // _NW


def _sigmoid(x):
    return 1.0 / (1.0 + jnp.exp(-x))


def _body(user_h, item_h, th_h, a_h, b_h, c_h, out_h,
          uidx, iidx, thv, av, bv, cv, outv, sem):
    wid = lax.axis_index("s") * _NC + lax.axis_index("c")
    base = wid * _CHUNK
    pltpu.sync_copy(user_h.at[pl.ds(base, _CHUNK)], uidx)
    pltpu.sync_copy(item_h.at[pl.ds(base, _CHUNK)], iidx)
    cps = [
        pltpu.async_copy(th_h.at[uidx], thv, sem),
        pltpu.async_copy(a_h.at[iidx], av, sem),
        pltpu.async_copy(b_h.at[iidx], bv, sem),
        pltpu.async_copy(c_h.at[iidx], cv, sem),
    ]
    for cp in cps:
        cp.wait()

    def step(i, carry):
        s = pl.ds(i * _L, _L)
        theta = VALUE_RANGE * (_sigmoid(thv[s]) - 0.5)
        b = VALUE_RANGE * (_sigmoid(bv[s]) - 0.5)
        a = A_RANGE * _sigmoid(av[s])
        c = _sigmoid(cv[s])
        outv[s] = c + (1.0 - c) / (1.0 + jnp.exp(-DCONST * a * (theta - b)))
        return carry

    lax.fori_loop(0, _CHUNK // _L, step, 0, unroll=4)
    pltpu.sync_copy(outv, out_h.at[pl.ds(base, _CHUNK)])


@jax.jit
def kernel(user, item, theta_w, a_w, b_w, c_w):
    run = pl.kernel(
        _body,
        out_type=jax.ShapeDtypeStruct((BATCH,), jnp.float32),
        mesh=plsc.VectorSubcoreMesh(core_axis_name="c", subcore_axis_name="s"),
        scratch_types=[
            pltpu.VMEM((_CHUNK,), jnp.int32),
            pltpu.VMEM((_CHUNK,), jnp.int32),
            pltpu.VMEM((_CHUNK,), jnp.float32),
            pltpu.VMEM((_CHUNK,), jnp.float32),
            pltpu.VMEM((_CHUNK,), jnp.float32),
            pltpu.VMEM((_CHUNK,), jnp.float32),
            pltpu.VMEM((_CHUNK,), jnp.float32),
            pltpu.SemaphoreType.DMA,
        ],
    )
    return run(
        user.astype(jnp.int32),
        item.astype(jnp.int32),
        theta_w.reshape(-1),
        a_w.reshape(-1),
        b_w.reshape(-1),
        c_w.reshape(-1),
    )


# speedup vs baseline: 3.5465x; 2.9384x over previous
import functools

import jax
import jax.numpy as jnp
from jax import lax
from jax.experimental import pallas as pl
from jax.experimental.pallas import tpu as pltpu
from jax.experimental.pallas import tpu_sc as plsc

BATCH = 16384
VALUE_RANGE = 8.0
A_RANGE = 3.0
DCONST = 1.702

_info = plsc.get_sparse_core_info()
_NC, _NS, _L = _info.num_cores, _info.num_subcores, _info.num_lanes
_NW = _NC * _NS
_CHUNK = BATCH // _NW


def _sigmoid(x):
    return 1.0 / (1.0 + jnp.exp(-x))


def _body(user_h, item_h, th_h, a_h, b_h, c_h, out_h,
          uidx, iidx, thv, av, bv, cv, outv, sem):
    wid = lax.axis_index("s") * _NC + lax.axis_index("c")
    base = wid * _CHUNK
    pltpu.sync_copy(user_h.at[pl.ds(base, _CHUNK)], uidx)
    pltpu.sync_copy(item_h.at[pl.ds(base, _CHUNK)], iidx)
    cps = [
        pltpu.async_copy(th_h.at[0].at[uidx], thv, sem),
        pltpu.async_copy(a_h.at[0].at[iidx], av, sem),
        pltpu.async_copy(b_h.at[0].at[iidx], bv, sem),
        pltpu.async_copy(c_h.at[0].at[iidx], cv, sem),
    ]
    for cp in cps:
        cp.wait()

    def step(i, carry):
        s = pl.ds(i * _L, _L)
        theta = VALUE_RANGE * (_sigmoid(thv[s]) - 0.5)
        b = VALUE_RANGE * (_sigmoid(bv[s]) - 0.5)
        a = A_RANGE * _sigmoid(av[s])
        c = _sigmoid(cv[s])
        outv[s] = c + (1.0 - c) / (1.0 + jnp.exp(-DCONST * a * (theta - b)))
        return carry

    lax.fori_loop(0, _CHUNK // _L, step, 0, unroll=4)
    pltpu.sync_copy(outv, out_h.at[pl.ds(base, _CHUNK)])


@jax.jit
def kernel(user, item, theta_w, a_w, b_w, c_w):
    run = pl.kernel(
        _body,
        out_type=jax.ShapeDtypeStruct((BATCH,), jnp.float32),
        mesh=plsc.VectorSubcoreMesh(core_axis_name="c", subcore_axis_name="s"),
        scratch_types=[
            pltpu.VMEM((_CHUNK,), jnp.int32),
            pltpu.VMEM((_CHUNK,), jnp.int32),
            pltpu.VMEM((_CHUNK,), jnp.float32),
            pltpu.VMEM((_CHUNK,), jnp.float32),
            pltpu.VMEM((_CHUNK,), jnp.float32),
            pltpu.VMEM((_CHUNK,), jnp.float32),
            pltpu.VMEM((_CHUNK,), jnp.float32),
            pltpu.SemaphoreType.DMA,
        ],
    )
    return run(
        user.astype(jnp.int32),
        item.astype(jnp.int32),
        theta_w.reshape(1, -1),
        a_w.reshape(1, -1),
        b_w.reshape(1, -1),
        c_w.reshape(1, -1),
    )


# speedup vs baseline: 3.5786x; 1.0091x over previous
import functools

import jax
import jax.numpy as jnp
from jax import lax
from jax.experimental import pallas as pl
from jax.experimental.pallas import tpu as pltpu
from jax.experimental.pallas import tpu_sc as plsc

BATCH = 16384
VALUE_RANGE = 8.0
A_RANGE = 3.0
DCONST = 1.702

_info = plsc.get_sparse_core_info()
_NC, _NS, _L = _info.num_cores, _info.num_subcores, _info.num_lanes
_NW = _NC * _NS
_CHUNK = BATCH // _NW


def _sigmoid(x):
    return 1.0 / (1.0 + jnp.exp(-x))


_HALF = _CHUNK // 2


def _body(user_h, item_h, th_h, a_h, b_h, c_h, out_h,
          uidx, iidx, thv, av, bv, cv, outv, semi, sema, semb, semo):
    wid = lax.axis_index("s") * _NC + lax.axis_index("c")
    base = wid * _CHUNK
    cpu = pltpu.async_copy(user_h.at[pl.ds(base, _CHUNK)], uidx, semi)
    cpi = pltpu.async_copy(item_h.at[pl.ds(base, _CHUNK)], iidx, semi)
    cpu.wait()
    cpi.wait()

    def fire(lo, sem):
        s = pl.ds(lo, _HALF)
        return [
            pltpu.async_copy(th_h.at[0].at[uidx.at[s]], thv.at[s], sem),
            pltpu.async_copy(a_h.at[0].at[iidx.at[s]], av.at[s], sem),
            pltpu.async_copy(b_h.at[0].at[iidx.at[s]], bv.at[s], sem),
            pltpu.async_copy(c_h.at[0].at[iidx.at[s]], cv.at[s], sem),
        ]

    ga = fire(0, sema)
    gb = fire(_HALF, semb)

    def step(i, carry):
        s = pl.ds(i * _L, _L)
        theta = VALUE_RANGE * (_sigmoid(thv[s]) - 0.5)
        b = VALUE_RANGE * (_sigmoid(bv[s]) - 0.5)
        a = A_RANGE * _sigmoid(av[s])
        c = _sigmoid(cv[s])
        outv[s] = c + (1.0 - c) / (1.0 + jnp.exp(-DCONST * a * (theta - b)))
        return carry

    nsteps = _HALF // _L
    for cp in ga:
        cp.wait()
    lax.fori_loop(0, nsteps, step, 0, unroll=4)
    oa = pltpu.async_copy(outv.at[pl.ds(0, _HALF)],
                          out_h.at[pl.ds(base, _HALF)], semo)
    for cp in gb:
        cp.wait()
    lax.fori_loop(nsteps, 2 * nsteps, step, 0, unroll=4)
    ob = pltpu.async_copy(outv.at[pl.ds(_HALF, _HALF)],
                          out_h.at[pl.ds(base + _HALF, _HALF)], semo)
    oa.wait()
    ob.wait()


@jax.jit
def kernel(user, item, theta_w, a_w, b_w, c_w):
    run = pl.kernel(
        _body,
        out_type=jax.ShapeDtypeStruct((BATCH,), jnp.float32),
        mesh=plsc.VectorSubcoreMesh(core_axis_name="c", subcore_axis_name="s"),
        scratch_types=[
            pltpu.VMEM((_CHUNK,), jnp.int32),
            pltpu.VMEM((_CHUNK,), jnp.int32),
            pltpu.VMEM((_CHUNK,), jnp.float32),
            pltpu.VMEM((_CHUNK,), jnp.float32),
            pltpu.VMEM((_CHUNK,), jnp.float32),
            pltpu.VMEM((_CHUNK,), jnp.float32),
            pltpu.VMEM((_CHUNK,), jnp.float32),
            pltpu.SemaphoreType.DMA,
            pltpu.SemaphoreType.DMA,
            pltpu.SemaphoreType.DMA,
            pltpu.SemaphoreType.DMA,
        ],
    )
    return run(
        user.astype(jnp.int32),
        item.astype(jnp.int32),
        theta_w.reshape(1, -1),
        a_w.reshape(1, -1),
        b_w.reshape(1, -1),
        c_w.reshape(1, -1),
    )
